# baseline (speedup 1.0000x reference)
import functools

import jax
import jax.numpy as jnp
from jax import lax
from jax.experimental import pallas as pl
from jax.experimental.pallas import tpu as pltpu
from jax.experimental.pallas import tpu_sc as plsc

D = 64
NC = 2
NS = 16
NW = NC * NS
FLD = 26
BB = 128
NBUF = 2


@jax.jit
def _sc_gather(idx_fmaj, table):
    batch = idx_fmaj.shape[0] // FLD
    nbc = batch // BB
    nblk = FLD * nbc
    blk_per_w = nblk // NW
    mesh = plsc.VectorSubcoreMesh(core_axis_name="c", subcore_axis_name="s")

    @functools.partial(
        pl.kernel,
        out_type=jax.ShapeDtypeStruct((FLD, 8, nbc, 8, BB), jnp.float32),
        mesh=mesh,
        scratch_types=[
            pltpu.VMEM((blk_per_w * BB,), jnp.int32),
            pltpu.VMEM((NBUF, BB, D), jnp.float32),
            pltpu.VMEM((NBUF, D, BB), jnp.float32),
            [pltpu.SemaphoreType.DMA] * NBUF,
            [pltpu.SemaphoreType.DMA] * NBUF,
        ],
        compiler_params=pltpu.CompilerParams(
            use_tc_tiling_on_sc=False, needs_layout_passes=False
        ),
    )
    def k(idx_hbm, tab_hbm, out_hbm, idx_v, rows_v, tr_v, gsems, wsems):
        wid = lax.axis_index("s") * NC + lax.axis_index("c")
        blk0 = wid * blk_per_w
        pltpu.sync_copy(
            idx_hbm.at[pl.ds(blk0 * BB, blk_per_w * BB)], idx_v
        )
        lane = lax.iota(jnp.int32, 16)

        def start_gather(kk, p):
            pltpu.async_copy(
                tab_hbm.at[idx_v.at[pl.ds(kk * BB, BB)]],
                rows_v.at[p],
                gsems[p],
            )

        def wait_gather(p):
            pltpu.make_async_copy(
                tab_hbm.at[idx_v.at[pl.ds(0, BB)]], rows_v.at[p], gsems[p]
            ).wait()

        def transpose(p):
            src = rows_v.at[p]
            dst = tr_v.at[p]
            for d in range(D):
                colv = jnp.full((16,), d, jnp.int32)
                for g in range(BB // 16):
                    v = plsc.load_gather(src, [lane + (16 * g), colv])
                    dst[d, pl.ds(16 * g, 16)] = v

        def start_write(kk, p):
            blk = blk0 + kk
            f = blk // nbc
            bc = blk % nbc
            for d8 in range(8):
                pltpu.async_copy(
                    tr_v.at[p].at[pl.ds(d8 * 8, 8)],
                    out_hbm.at[f, d8, bc],
                    wsems[p],
                )

        def wait_write(p):
            for d8 in range(8):
                pltpu.make_async_copy(
                    tr_v.at[p].at[pl.ds(d8 * 8, 8)],
                    out_hbm.at[0, 0, 0],
                    wsems[p],
                ).wait()

        for p in range(NBUF):
            start_gather(p, p)

        def body(g, carry):
            for p in range(NBUF):
                kk = g * NBUF + p
                wait_gather(p)

                @pl.when(g > 0)
                def _():
                    wait_write(p)

                transpose(p)
                start_write(kk, p)

                @pl.when(kk + NBUF < blk_per_w)
                def _():
                    start_gather(kk + NBUF, p)

            return carry

        lax.fori_loop(0, blk_per_w // NBUF, body, 0, unroll=False)
        for p in range(NBUF):
            wait_write(p)

    return k(idx_fmaj, table)


def kernel(category_ids, embedding_table):
    batch, fields = category_ids.shape
    idx_fmaj = category_ids.T.reshape(batch * fields).astype(jnp.int32)
    out5 = _sc_gather(idx_fmaj, embedding_table)
    return out5.transpose((2, 4, 0, 1, 3)).reshape(batch, fields, D)


# speedup vs baseline: 1.4782x; 1.4782x over previous
import functools

import jax
import jax.numpy as jnp
from jax import lax
from jax.experimental import pallas as pl
from jax.experimental.pallas import tpu as pltpu
from jax.experimental.pallas import tpu_sc as plsc

D = 64
NC = 2
NS = 16
NW = NC * NS
FLD = 26
BB = 128
NBUF = 2


@jax.jit
def _sc_gather(idx_fmaj, table):
    batch = idx_fmaj.shape[0] // FLD
    nbc = batch // BB
    nblk = FLD * nbc
    blk_per_w = nblk // NW
    mesh = plsc.VectorSubcoreMesh(core_axis_name="c", subcore_axis_name="s")

    @functools.partial(
        pl.kernel,
        out_type=jax.ShapeDtypeStruct((FLD, 8, nbc, 8, BB), jnp.float32),
        mesh=mesh,
        scratch_types=[
            pltpu.VMEM((blk_per_w * BB,), jnp.int32),
            pltpu.VMEM((NBUF, BB, D), jnp.float32),
            pltpu.VMEM((NBUF, D, BB), jnp.float32),
            [pltpu.SemaphoreType.DMA] * NBUF,
            [pltpu.SemaphoreType.DMA] * NBUF,
        ],
        compiler_params=pltpu.CompilerParams(
            use_tc_tiling_on_sc=False, needs_layout_passes=False
        ),
    )
    def k(idx_hbm, tab_hbm, out_hbm, idx_v, rows_v, tr_v, gsems, wsems):
        wid = lax.axis_index("s") * NC + lax.axis_index("c")
        blk0 = wid * blk_per_w
        pltpu.sync_copy(
            idx_hbm.at[pl.ds(blk0 * BB, blk_per_w * BB)], idx_v
        )
        lane = lax.iota(jnp.int32, 16)

        def start_gather(kk, p):
            pltpu.async_copy(
                tab_hbm.at[idx_v.at[pl.ds(kk * BB, BB)]],
                rows_v.at[p],
                gsems[p],
            )

        def wait_gather(p):
            pltpu.make_async_copy(
                tab_hbm.at[idx_v.at[pl.ds(0, BB)]], rows_v.at[p], gsems[p]
            ).wait()

        def transpose(p):
            src = rows_v.at[p]
            dst = tr_v.at[p]

            def step(i, carry):
                rowv = lane + i * 16
                for d0 in range(0, D, 16):
                    for s in range(16):
                        perm = lax.bitwise_and(lane + s, 15) + d0
                        v = plsc.load_gather(src, [rowv, perm])
                        plsc.store_scatter(dst, [perm, rowv], v)
                return carry

            lax.fori_loop(0, BB // 16, step, 0, unroll=False)

        def start_write(kk, p):
            blk = blk0 + kk
            f = blk // nbc
            bc = blk % nbc
            for d8 in range(8):
                pltpu.async_copy(
                    tr_v.at[p].at[pl.ds(d8 * 8, 8)],
                    out_hbm.at[f, d8, bc],
                    wsems[p],
                )

        def wait_write(p):
            for d8 in range(8):
                pltpu.make_async_copy(
                    tr_v.at[p].at[pl.ds(d8 * 8, 8)],
                    out_hbm.at[0, 0, 0],
                    wsems[p],
                ).wait()

        for p in range(NBUF):
            start_gather(p, p)

        def body(g, carry):
            for p in range(NBUF):
                kk = g * NBUF + p
                wait_gather(p)

                @pl.when(g > 0)
                def _():
                    wait_write(p)

                transpose(p)
                start_write(kk, p)

                @pl.when(kk + NBUF < blk_per_w)
                def _():
                    start_gather(kk + NBUF, p)

            return carry

        lax.fori_loop(0, blk_per_w // NBUF, body, 0, unroll=False)
        for p in range(NBUF):
            wait_write(p)

    return k(idx_fmaj, table)


def kernel(category_ids, embedding_table):
    batch, fields = category_ids.shape
    idx_fmaj = category_ids.T.reshape(batch * fields).astype(jnp.int32)
    out5 = _sc_gather(idx_fmaj, embedding_table)
    return out5.transpose((2, 4, 0, 1, 3)).reshape(batch, fields, D)


# speedup vs baseline: 1.7642x; 1.1935x over previous
import functools

import jax
import jax.numpy as jnp
from jax import lax
from jax.experimental import pallas as pl
from jax.experimental.pallas import tpu as pltpu
from jax.experimental.pallas import tpu_sc as plsc

D = 64
NC = 2
NS = 16
NW = NC * NS
FLD = 26
BB = 128
NBUF = 2
V = 1000000
NSLAB = V // 128
VTAIL = NSLAB * 128


def _diag(lane, s):
    return lax.bitwise_and(lane + s, 15)


@jax.jit
def _sc_pairs(tab_t, tab_tail):
    mesh = plsc.VectorSubcoreMesh(core_axis_name="c", subcore_axis_name="s")
    slab_max = (NSLAB + NW - 1) // NW

    @functools.partial(
        pl.kernel,
        out_type=jax.ShapeDtypeStruct((V // 2, 128), jnp.float32),
        mesh=mesh,
        scratch_types=[
            pltpu.VMEM((NBUF, D, 128), jnp.float32),
            pltpu.VMEM((NBUF, D, 128), jnp.float32),
            pltpu.VMEM((D, D), jnp.float32),
            pltpu.VMEM((32, 128), jnp.float32),
            [pltpu.SemaphoreType.DMA] * NBUF,
            [pltpu.SemaphoreType.DMA] * NBUF,
        ],
        compiler_params=pltpu.CompilerParams(
            use_tc_tiling_on_sc=True, needs_layout_passes=False
        ),
    )
    def k(tabt_hbm, tail_hbm, pairs_hbm, slab_v, pb_v, tsl_v, tpr_v,
          gsems, wsems):
        wid = lax.axis_index("s") * NC + lax.axis_index("c")
        lane = lax.iota(jnp.int32, 16)

        def slab_of(t):
            return wid + NW * t

        def start_in(t, p):
            c = slab_of(t)

            @pl.when(c < NSLAB)
            def _():
                pltpu.async_copy(
                    tabt_hbm.at[:, pl.ds(c * 128, 128)],
                    slab_v.at[p],
                    gsems[p],
                )

        def wait_in(t, p):
            c = slab_of(t)

            @pl.when(c < NSLAB)
            def _():
                pltpu.make_async_copy(
                    tabt_hbm.at[:, pl.ds(0, 128)], slab_v.at[p], gsems[p]
                ).wait()

        def transpose_pairs(p):
            src = slab_v.at[p]
            dst = pb_v.at[p]

            def step(ij, carry):
                i = ij // 4
                j0 = (ij % 4) * 16
                q0 = i * 16
                rowv = lax.bitwise_and(q0 + lane, 63)
                hp = lax.shift_right_logical(q0 + lane, 6)
                for s in range(16):
                    jj = j0 + _diag(lane, s)
                    colv = jj * 2 + hp
                    v = plsc.load_gather(src, [rowv, colv])
                    plsc.store_scatter(dst, [jj, q0 + lane], v)
                return carry

            lax.fori_loop(0, 32, step, 0, unroll=False)

        def start_out(t, p):
            c = slab_of(t)

            @pl.when(c < NSLAB)
            def _():
                pltpu.async_copy(
                    pb_v.at[p], pairs_hbm.at[pl.ds(c * D, D)], wsems[p]
                )

        def wait_out(t, p):
            c = slab_of(t)

            @pl.when(c < NSLAB)
            def _():
                pltpu.make_async_copy(
                    pb_v.at[p], pairs_hbm.at[pl.ds(0, D)], wsems[p]
                ).wait()

        for p in range(NBUF):
            start_in(p, p)

        def body(g, carry):
            for p in range(NBUF):
                t = g * NBUF + p
                wait_in(t, p)

                @pl.when(g > 0)
                def _():
                    wait_out(t - NBUF, p)

                @pl.when(slab_of(t) < NSLAB)
                def _():
                    transpose_pairs(p)

                start_out(t, p)
                start_in(t + NBUF, p)
            return carry

        nloop = (slab_max + NBUF - 1) // NBUF
        lax.fori_loop(0, nloop, body, 0, unroll=False)
        for p in range(NBUF):
            wait_out(nloop * NBUF - NBUF + p, p)

        @pl.when(wid == 0)
        def _():
            pltpu.sync_copy(tail_hbm, tsl_v)

            def tstep(ij, carry):
                q0 = (ij // 2) * 16
                j0 = (ij % 2) * 16
                rowv = lax.bitwise_and(q0 + lane, 63)
                hp = lax.shift_right_logical(q0 + lane, 6)
                for s in range(16):
                    jj = j0 + _diag(lane, s)
                    colv = jj * 2 + hp
                    v = plsc.load_gather(tsl_v, [rowv, colv])
                    plsc.store_scatter(tpr_v, [jj, q0 + lane], v)
                return carry

            lax.fori_loop(0, 16, tstep, 0, unroll=False)
            pltpu.sync_copy(tpr_v, pairs_hbm.at[pl.ds(VTAIL // 2, 32)])

    return k(tab_t, tab_tail)


@jax.jit
def _sc_gather(idx_fmaj, pairs):
    batch = idx_fmaj.shape[0] // FLD
    nbc = batch // BB
    nblk = FLD * nbc
    blk_per_w = nblk // NW
    n_idx = blk_per_w * BB
    mesh = plsc.VectorSubcoreMesh(core_axis_name="c", subcore_axis_name="s")

    @functools.partial(
        pl.kernel,
        out_type=jax.ShapeDtypeStruct((FLD, 8, nbc, 8, BB), jnp.float32),
        mesh=mesh,
        scratch_types=[
            pltpu.VMEM((n_idx,), jnp.int32),
            pltpu.VMEM((n_idx,), jnp.int32),
            pltpu.VMEM((NBUF, BB, 128), jnp.float32),
            pltpu.VMEM((NBUF, D, BB), jnp.float32),
            [pltpu.SemaphoreType.DMA] * NBUF,
            [pltpu.SemaphoreType.DMA] * NBUF,
        ],
        compiler_params=pltpu.CompilerParams(
            use_tc_tiling_on_sc=True, needs_layout_passes=False
        ),
    )
    def k(idx_hbm, pairs_hbm, out_hbm, idx_v, jdx_v, rows_v, tr_v,
          gsems, wsems):
        wid = lax.axis_index("s") * NC + lax.axis_index("c")
        blk0 = wid * blk_per_w
        pltpu.sync_copy(idx_hbm.at[pl.ds(blk0 * BB, n_idx)], idx_v)
        lane = lax.iota(jnp.int32, 16)

        def mkpairs(i, carry):
            v = idx_v[pl.ds(i * 16, 16)]
            jdx_v[pl.ds(i * 16, 16)] = lax.shift_right_logical(v, 1)
            return carry

        lax.fori_loop(0, n_idx // 16, mkpairs, 0, unroll=False)

        def start_gather(kk, p):
            pltpu.async_copy(
                pairs_hbm.at[jdx_v.at[pl.ds(kk * BB, BB)]],
                rows_v.at[p],
                gsems[p],
            )

        def wait_gather(p):
            pltpu.make_async_copy(
                pairs_hbm.at[jdx_v.at[pl.ds(0, BB)]], rows_v.at[p], gsems[p]
            ).wait()

        def transpose(kk, p):
            src = rows_v.at[p]
            dst = tr_v.at[p]

            def step(ij, carry):
                i = ij // 4
                d0 = (ij % 4) * 16
                rowv = lane + i * 16
                iv = idx_v[pl.ds(kk * BB + i * 16, 16)]
                hv64 = lax.shift_left(lax.bitwise_and(iv, 1), 6)
                for s in range(16):
                    perm = _diag(lane, s) + d0
                    v = plsc.load_gather(src, [rowv, perm + hv64])
                    plsc.store_scatter(dst, [perm, rowv], v)
                return carry

            lax.fori_loop(0, (BB // 16) * 4, step, 0, unroll=False)

        def start_write(kk, p):
            blk = blk0 + kk
            f = blk // nbc
            bc = blk % nbc
            for d8 in range(8):
                pltpu.async_copy(
                    tr_v.at[p].at[pl.ds(d8 * 8, 8)],
                    out_hbm.at[f, d8, bc],
                    wsems[p],
                )

        def wait_write(p):
            for d8 in range(8):
                pltpu.make_async_copy(
                    tr_v.at[p].at[pl.ds(d8 * 8, 8)],
                    out_hbm.at[0, 0, 0],
                    wsems[p],
                ).wait()

        for p in range(NBUF):
            start_gather(p, p)

        def body(g, carry):
            for p in range(NBUF):
                kk = g * NBUF + p
                wait_gather(p)

                @pl.when(g > 0)
                def _():
                    wait_write(p)

                transpose(kk, p)
                start_write(kk, p)

                @pl.when(kk + NBUF < blk_per_w)
                def _():
                    start_gather(kk + NBUF, p)

            return carry

        lax.fori_loop(0, blk_per_w // NBUF, body, 0, unroll=False)
        for p in range(NBUF):
            wait_write(p)

    return k(idx_fmaj, pairs)


def kernel(category_ids, embedding_table):
    batch, fields = category_ids.shape
    idx_fmaj = category_ids.T.reshape(batch * fields).astype(jnp.int32)
    tab_t = embedding_table.T
    tab_tail = embedding_table[VTAIL:].T
    pairs = _sc_pairs(tab_t, tab_tail)
    out5 = _sc_gather(idx_fmaj, pairs)
    return out5.transpose((2, 4, 0, 1, 3)).reshape(batch, fields, D)
